# baseline (speedup 1.0000x reference)
import jax
import jax.numpy as jnp
from jax.experimental import pallas as pl
from jax.experimental.pallas import tpu as pltpu

BM = 2048
CK = 512
BN = 512


def _mm_body(x_ref, w_ref, b_ref, o_ref, xb_ref):
    t = pl.program_id(1)
    nc = xb_ref.shape[1] // x_ref.shape[1]

    @pl.when(t < nc)
    def _():
        base = pl.multiple_of(t * CK, CK)
        xb_ref[:, pl.ds(base, CK)] = x_ref[...].astype(jnp.bfloat16)

    @pl.when(t >= nc)
    def _():
        acc = jax.lax.dot_general(
            xb_ref[...],
            w_ref[...].astype(jnp.bfloat16),
            dimension_numbers=(((1,), (1,)), ((), ())),
            preferred_element_type=jnp.float32,
        )
        o_ref[...] = acc + b_ref[...]


@jax.jit
def kernel(x, W, b):
    M, K = x.shape
    N = W.shape[0]
    nc = K // CK
    nd = N // BN
    b2 = b.reshape(1, N)
    out = pl.pallas_call(
        _mm_body,
        grid=(M // BM, nc + nd),
        in_specs=[
            pl.BlockSpec(
                (BM, CK), lambda i, t: (i, jnp.where(t < nc, t, nc - 1))
            ),
            pl.BlockSpec(
                (BN, K), lambda i, t: (jnp.where(t >= nc, t - nc, 0), 0)
            ),
            pl.BlockSpec(
                (1, BN), lambda i, t: (0, jnp.where(t >= nc, t - nc, 0))
            ),
        ],
        out_specs=pl.BlockSpec(
            (BM, BN), lambda i, t: (i, jnp.where(t >= nc, t - nc, 0))
        ),
        out_shape=jax.ShapeDtypeStruct((M, N), jnp.float32),
        scratch_shapes=[pltpu.VMEM((BM, K), jnp.bfloat16)],
        compiler_params=pltpu.CompilerParams(
            dimension_semantics=("arbitrary", "arbitrary"),
            vmem_limit_bytes=100 * 1024 * 1024,
        ),
    )(x, W, b2)
    return out


# speedup vs baseline: 1.0162x; 1.0162x over previous
import jax
import jax.numpy as jnp
from jax.experimental import pallas as pl
from jax.experimental.pallas import tpu as pltpu

BM = 2048
CK = 1024
BN = 512


def _mm_body(x_ref, w_ref, b_ref, o_ref, xb_ref):
    t = pl.program_id(1)
    nc = xb_ref.shape[1] // x_ref.shape[1]

    @pl.when(t < nc)
    def _():
        base = pl.multiple_of(t * CK, CK)
        xb_ref[:, pl.ds(base, CK)] = x_ref[...].astype(jnp.bfloat16)

    @pl.when(t >= nc)
    def _():
        acc = jax.lax.dot_general(
            xb_ref[...],
            w_ref[...].astype(jnp.bfloat16),
            dimension_numbers=(((1,), (1,)), ((), ())),
            preferred_element_type=jnp.float32,
        )
        o_ref[...] = acc + b_ref[...]


@jax.jit
def kernel(x, W, b):
    M, K = x.shape
    N = W.shape[0]
    nc = K // CK
    nd = N // BN
    b2 = b.reshape(1, N)
    out = pl.pallas_call(
        _mm_body,
        grid=(M // BM, nc + nd),
        in_specs=[
            pl.BlockSpec(
                (BM, CK), lambda i, t: (i, jnp.where(t < nc, t, nc - 1))
            ),
            pl.BlockSpec(
                (BN, K), lambda i, t: (jnp.where(t >= nc, t - nc, 0), 0)
            ),
            pl.BlockSpec(
                (1, BN), lambda i, t: (0, jnp.where(t >= nc, t - nc, 0))
            ),
        ],
        out_specs=pl.BlockSpec(
            (BM, BN), lambda i, t: (i, jnp.where(t >= nc, t - nc, 0))
        ),
        out_shape=jax.ShapeDtypeStruct((M, N), jnp.float32),
        scratch_shapes=[pltpu.VMEM((BM, K), jnp.bfloat16)],
        compiler_params=pltpu.CompilerParams(
            dimension_semantics=("arbitrary", "arbitrary"),
            vmem_limit_bytes=100 * 1024 * 1024,
        ),
    )(x, W, b2)
    return out
